# baseline (speedup 1.0000x reference)
import functools

import jax
import jax.numpy as jnp
from jax import lax
from jax.experimental import pallas as pl
from jax.experimental.pallas import tpu as pltpu
from jax.experimental.pallas import tpu_sc as plsc

N = 10000
E = 320000
D_IN = 256
H = 16

NC = 2
NS = 16
NW = NC * NS
L = 16

EPAD = 327680
NROWS = EPAD // 128
ROWS_PT = NROWS // NS
ROWS_PC = 8
CHUNKS = ROWS_PT // ROWS_PC
CE = ROWS_PC * 128
NACC = 10240
SLICE = NACC // NS

_mesh = plsc.VectorSubcoreMesh(core_axis_name="c", subcore_axis_name="s",
                               num_cores=1)
_sc_params = pltpu.CompilerParams(use_tc_tiling_on_sc=False)


@functools.partial(
    pl.kernel,
    out_type=jax.ShapeDtypeStruct((1, NACC), jnp.float32),
    mesh=_mesh,
    scratch_types=[
        pltpu.VMEM((ROWS_PT, 2, 128), jnp.int32),
        pltpu.VMEM((128,), jnp.float32),
        pltpu.VMEM((SLICE,), jnp.float32),
        pltpu.VMEM_SHARED((NACC,), jnp.float32),
        pltpu.SemaphoreType.DMA,
        pltpu.SemaphoreType.DMA,
    ],
    compiler_params=_sc_params,
)
def _k_deg(sd_hbm, deg_out_hbm, sd_v, ones_v, zbuf, deg_sh, sem_t, sem):
    sid = lax.axis_index("s")
    zero = jnp.zeros((L,), jnp.float32)
    one = jnp.ones((L,), jnp.float32)

    t = pltpu.async_copy(sd_hbm.at[pl.ds(sid * ROWS_PT, ROWS_PT)], sd_v, sem_t)

    def zbody(i, _):
        zbuf[pl.ds(i * L, L)] = zero
        return 0
    lax.fori_loop(0, SLICE // L, zbody, 0)
    for j in range(128 // L):
        ones_v[pl.ds(j * L, L)] = one
    pltpu.sync_copy(zbuf, deg_sh.at[pl.ds(sid * SLICE, SLICE)])
    t.wait()
    plsc.subcore_barrier()
    descs = [
        pltpu.async_copy(ones_v, deg_sh.at[sd_v.at[j].at[1]], sem, add=True)
        for j in range(ROWS_PT)
    ]
    for d in descs:
        d.wait()

    plsc.subcore_barrier()
    pltpu.sync_copy(deg_sh.at[pl.ds(sid * SLICE, SLICE)],
                    deg_out_hbm.at[0].at[pl.ds(sid * SLICE, SLICE)])


NBUF = 4


@functools.partial(
    pl.kernel,
    out_type=jax.ShapeDtypeStruct((NACC, H), jnp.float32),
    mesh=_mesh,
    scratch_types=[
        pltpu.VMEM((NBUF, ROWS_PC, 2, 128), jnp.int32),
        pltpu.VMEM((NBUF, CE, H), jnp.float32),
        pltpu.VMEM((SLICE, H), jnp.float32),
        pltpu.VMEM_SHARED((NACC, H), jnp.float32),
    ] + [pltpu.SemaphoreType.DMA] * (3 * NBUF),
    compiler_params=_sc_params,
)
def _k_scat(xs_hbm, sd_hbm, acc_out_hbm, sd_v, rows_v, zbuf, acc_sh, *sems):
    sid = lax.axis_index("s")
    zero = jnp.zeros((L,), jnp.float32)
    sems_t = sems[0:NBUF]
    sems_g = sems[NBUF:2 * NBUF]
    sems_s = sems[2 * NBUF:3 * NBUF]

    def run(base, nchunks):
        def fire_stage(g):
            b = g % NBUF
            return pltpu.async_copy(
                sd_hbm.at[pl.ds(base + g * ROWS_PC, ROWS_PC)],
                sd_v.at[b], sems_t[b])

        def fire_gath(g):
            b = g % NBUF
            return [
                pltpu.async_copy(xs_hbm.at[sd_v.at[b].at[j].at[0]],
                                 rows_v.at[b].at[pl.ds(j * 128, 128)],
                                 sems_g[b])
                for j in range(ROWS_PC)
            ]

        def fire_scat(g):
            b = g % NBUF
            return [
                pltpu.async_copy(rows_v.at[b].at[pl.ds(j * 128, 128)],
                                 acc_sh.at[sd_v.at[b].at[j].at[1]],
                                 sems_s[b], add=True)
                for j in range(ROWS_PC)
            ]

        tdescs, gdescs, sdescs = {}, {}, {}
        for g in range(min(2, nchunks)):
            tdescs[g] = fire_stage(g)
        tdescs.pop(0).wait()
        gdescs[0] = fire_gath(0)

        def zbody(i, _):
            zbuf[i] = zero
            return 0
        lax.fori_loop(0, SLICE, zbody, 0)
        pltpu.sync_copy(zbuf, acc_sh.at[pl.ds(sid * SLICE, SLICE)])
        plsc.subcore_barrier()

        for g in range(nchunks):
            if g - 2 >= 0:
                for d in sdescs.pop(g - 2):
                    d.wait()
            if g + 2 < nchunks:
                tdescs[g + 2] = fire_stage(g + 2)
            if g + 1 < nchunks:
                tdescs.pop(g + 1).wait()
                gdescs[g + 1] = fire_gath(g + 1)
            for d in gdescs.pop(g):
                d.wait()
            sdescs[g] = fire_scat(g)
        for g in (nchunks - 2, nchunks - 1):
            for d in sdescs.pop(g, []):
                d.wait()

    run(sid * ROWS_PT, CHUNKS)

    plsc.subcore_barrier()
    pltpu.sync_copy(acc_sh.at[pl.ds(sid * SLICE, SLICE)],
                    acc_out_hbm.at[pl.ds(sid * SLICE, SLICE)])


def _k_dinv_body(deg_ref, dinv_ref):
    deg = deg_ref[0, :] + 1.0
    dinv = lax.rsqrt(deg).reshape(deg.shape[0], 1)
    dinv_ref[...] = jnp.broadcast_to(dinv, dinv_ref.shape)


def _dinv_call(deg_parts):
    blk = 1024
    return pl.pallas_call(
        _k_dinv_body,
        grid=(NACC // blk,),
        in_specs=[pl.BlockSpec((1, blk), lambda i: (0, i))],
        out_specs=pl.BlockSpec((blk, H), lambda i: (i, 0)),
        out_shape=jax.ShapeDtypeStruct((NACC, H), jnp.float32),
    )(deg_parts)


def _k_dense1_body(x_ref, w_ref, dinv_ref, xs_ref):
    xw = jnp.dot(x_ref[...], w_ref[...], preferred_element_type=jnp.float32)
    xs_ref[...] = xw * dinv_ref[...]


def _dense1_call(x, W1, dinv):
    blk = 1000
    return pl.pallas_call(
        _k_dense1_body,
        grid=(N // blk,),
        in_specs=[
            pl.BlockSpec((blk, D_IN), lambda i: (i, 0)),
            pl.BlockSpec((D_IN, H), lambda i: (0, 0)),
            pl.BlockSpec((blk, H), lambda i: (i, 0)),
        ],
        out_specs=pl.BlockSpec((blk, H), lambda i: (i, 0)),
        out_shape=jax.ShapeDtypeStruct((N, H), jnp.float32),
    )(x, W1, dinv)


def _k_dense2_body(acc_ref, xs1_ref, dinv_ref, b1_ref, w2_ref, xs2_ref):
    dinv = dinv_ref[...]
    h = dinv * (acc_ref[...] + xs1_ref[...]) + b1_ref[...]
    h = jnp.maximum(h, 0.0)
    xs2_ref[...] = jnp.dot(h, w2_ref[...],
                           preferred_element_type=jnp.float32) * dinv


def _dense2_call(acc, xs1, dinv, b1, W2):
    blk = 1000
    return pl.pallas_call(
        _k_dense2_body,
        grid=(N // blk,),
        in_specs=[
            pl.BlockSpec((blk, H), lambda i: (i, 0)),
            pl.BlockSpec((blk, H), lambda i: (i, 0)),
            pl.BlockSpec((blk, H), lambda i: (i, 0)),
            pl.BlockSpec((1, H), lambda i: (0, 0)),
            pl.BlockSpec((H, H), lambda i: (0, 0)),
        ],
        out_specs=pl.BlockSpec((blk, H), lambda i: (i, 0)),
        out_shape=jax.ShapeDtypeStruct((N, H), jnp.float32),
    )(acc, xs1, dinv, b1, W2)


def _k_dense3_body(acc_ref, xs2_ref, dinv_ref, b2_ref, out_ref):
    out_ref[...] = dinv_ref[...] * (acc_ref[...] + xs2_ref[...]) + b2_ref[...]


def _dense3_call(acc, xs2, dinv, b2):
    blk = 1000
    return pl.pallas_call(
        _k_dense3_body,
        grid=(N // blk,),
        in_specs=[
            pl.BlockSpec((blk, H), lambda i: (i, 0)),
            pl.BlockSpec((blk, H), lambda i: (i, 0)),
            pl.BlockSpec((blk, H), lambda i: (i, 0)),
            pl.BlockSpec((1, H), lambda i: (0, 0)),
        ],
        out_specs=pl.BlockSpec((blk, H), lambda i: (i, 0)),
        out_shape=jax.ShapeDtypeStruct((N, H), jnp.float32),
    )(acc, xs2, dinv, b2)


def kernel(x, coo, W1, b1, W2, b2):
    src = coo[:, 0]
    dst = coo[:, 1]
    npad = EPAD - E
    src_p = jnp.concatenate([src, jnp.zeros((npad,), jnp.int32)])
    dst_p = jnp.concatenate([dst, jnp.full((npad,), N, jnp.int32)])
    sd2d = jnp.stack([src_p.reshape(NROWS, 128),
                      dst_p.reshape(NROWS, 128)], axis=1)

    deg_parts = _k_deg(sd2d)
    dinv = _dinv_call(deg_parts)

    xs1 = _dense1_call(x, W1, dinv)
    acc1 = _k_scat(xs1, sd2d)
    xs2 = _dense2_call(acc1, xs1, dinv, b1.reshape(1, H), W2)
    acc2 = _k_scat(xs2, sd2d)
    out = _dense3_call(acc2, xs2, dinv, b2.reshape(1, H))
    return out


# speedup vs baseline: 1.1297x; 1.1297x over previous
import functools

import jax
import jax.numpy as jnp
from jax import lax
from jax.experimental import pallas as pl
from jax.experimental.pallas import tpu as pltpu
from jax.experimental.pallas import tpu_sc as plsc

N = 10000
E = 320000
D_IN = 256
H = 16

NC = 2
NS = 16
NW = NC * NS
L = 16

EPAD = 327680
NROWS = EPAD // 128
R0 = 144
R1 = NROWS // NS - R0
ROWS_PC = 8
CE = ROWS_PC * 128
NACC = 10240
SLICE = NACC // NS

_mesh = plsc.VectorSubcoreMesh(core_axis_name="c", subcore_axis_name="s")
_sc_params = pltpu.CompilerParams(use_tc_tiling_on_sc=False)


@functools.partial(
    pl.kernel,
    out_type=jax.ShapeDtypeStruct((NC, NACC), jnp.float32),
    mesh=_mesh,
    scratch_types=[
        pltpu.VMEM((R0, 2, 128), jnp.int32),
        pltpu.VMEM((128,), jnp.float32),
        pltpu.VMEM((SLICE,), jnp.float32),
        pltpu.VMEM_SHARED((NACC,), jnp.float32),
        pltpu.SemaphoreType.DMA,
        pltpu.SemaphoreType.DMA,
    ],
    compiler_params=_sc_params,
)
def _k_deg(sd_hbm, deg_out_hbm, sd_v, ones_v, zbuf, deg_sh, sem_t, sem):
    cid = lax.axis_index("c")
    sid = lax.axis_index("s")
    zero = jnp.zeros((L,), jnp.float32)
    one = jnp.ones((L,), jnp.float32)

    def run(base, nrows):
        t = pltpu.async_copy(sd_hbm.at[pl.ds(base, nrows)],
                             sd_v.at[pl.ds(0, nrows)], sem_t)

        def zbody(i, _):
            zbuf[pl.ds(i * L, L)] = zero
            return 0
        lax.fori_loop(0, SLICE // L, zbody, 0)
        for j in range(128 // L):
            ones_v[pl.ds(j * L, L)] = one
        pltpu.sync_copy(zbuf, deg_sh.at[pl.ds(sid * SLICE, SLICE)])
        t.wait()
        plsc.subcore_barrier()
        descs = [
            pltpu.async_copy(ones_v, deg_sh.at[sd_v.at[j].at[1]], sem,
                             add=True)
            for j in range(nrows)
        ]
        for d in descs:
            d.wait()

    @pl.when(cid == 0)
    def _():
        run(sid * R0, R0)

    @pl.when(cid == 1)
    def _():
        run(NS * R0 + sid * R1, R1)

    plsc.subcore_barrier()
    pltpu.sync_copy(deg_sh.at[pl.ds(sid * SLICE, SLICE)],
                    deg_out_hbm.at[cid].at[pl.ds(sid * SLICE, SLICE)])


NBUF = 4


@functools.partial(
    pl.kernel,
    out_type=jax.ShapeDtypeStruct((NC, NACC, H), jnp.float32),
    mesh=_mesh,
    scratch_types=[
        pltpu.VMEM((NBUF, ROWS_PC, 2, 128), jnp.int32),
        pltpu.VMEM((NBUF, CE, H), jnp.float32),
        pltpu.VMEM((SLICE, H), jnp.float32),
        pltpu.VMEM_SHARED((NACC, H), jnp.float32),
    ] + [pltpu.SemaphoreType.DMA] * (3 * NBUF),
    compiler_params=_sc_params,
)
def _k_scat(xs_hbm, sd_hbm, acc_out_hbm, sd_v, rows_v, zbuf, acc_sh, *sems):
    cid = lax.axis_index("c")
    sid = lax.axis_index("s")
    zero = jnp.zeros((L,), jnp.float32)
    sems_t = sems[0:NBUF]
    sems_g = sems[NBUF:2 * NBUF]
    sems_s = sems[2 * NBUF:3 * NBUF]

    def run(base, nchunks):
        def fire_stage(g):
            b = g % NBUF
            return pltpu.async_copy(
                sd_hbm.at[pl.ds(base + g * ROWS_PC, ROWS_PC)],
                sd_v.at[b], sems_t[b])

        def fire_gath(g):
            b = g % NBUF
            return [
                pltpu.async_copy(xs_hbm.at[sd_v.at[b].at[j].at[0]],
                                 rows_v.at[b].at[pl.ds(j * 128, 128)],
                                 sems_g[b])
                for j in range(ROWS_PC)
            ]

        def fire_scat(g):
            b = g % NBUF
            return [
                pltpu.async_copy(rows_v.at[b].at[pl.ds(j * 128, 128)],
                                 acc_sh.at[sd_v.at[b].at[j].at[1]],
                                 sems_s[b], add=True)
                for j in range(ROWS_PC)
            ]

        tdescs, gdescs, sdescs = {}, {}, {}
        for g in range(min(2, nchunks)):
            tdescs[g] = fire_stage(g)
        tdescs.pop(0).wait()
        gdescs[0] = fire_gath(0)

        def zbody(i, _):
            zbuf[i] = zero
            return 0
        lax.fori_loop(0, SLICE, zbody, 0)
        pltpu.sync_copy(zbuf, acc_sh.at[pl.ds(sid * SLICE, SLICE)])
        plsc.subcore_barrier()

        for g in range(nchunks):
            if g - 2 >= 0:
                for d in sdescs.pop(g - 2):
                    d.wait()
            if g + 2 < nchunks:
                tdescs[g + 2] = fire_stage(g + 2)
            if g + 1 < nchunks:
                tdescs.pop(g + 1).wait()
                gdescs[g + 1] = fire_gath(g + 1)
            for d in gdescs.pop(g):
                d.wait()
            sdescs[g] = fire_scat(g)
        for g in (nchunks - 2, nchunks - 1):
            for d in sdescs.pop(g, []):
                d.wait()

    @pl.when(cid == 0)
    def _():
        run(sid * R0, R0 // ROWS_PC)

    @pl.when(cid == 1)
    def _():
        run(NS * R0 + sid * R1, R1 // ROWS_PC)

    plsc.subcore_barrier()
    pltpu.sync_copy(acc_sh.at[pl.ds(sid * SLICE, SLICE)],
                    acc_out_hbm.at[cid].at[pl.ds(sid * SLICE, SLICE)])


def _k_dinv_body(deg_ref, dinv_ref):
    deg = deg_ref[0, :] + deg_ref[1, :] + 1.0
    dinv = lax.rsqrt(deg).reshape(deg.shape[0], 1)
    dinv_ref[...] = jnp.broadcast_to(dinv, dinv_ref.shape)


def _dinv_call(deg_parts):
    blk = 1024
    return pl.pallas_call(
        _k_dinv_body,
        grid=(NACC // blk,),
        in_specs=[pl.BlockSpec((NC, blk), lambda i: (0, i))],
        out_specs=pl.BlockSpec((blk, H), lambda i: (i, 0)),
        out_shape=jax.ShapeDtypeStruct((NACC, H), jnp.float32),
    )(deg_parts)


def _k_dense1_body(x_ref, w_ref, dinv_ref, xs_ref):
    xw = jnp.dot(x_ref[...], w_ref[...], preferred_element_type=jnp.float32)
    xs_ref[...] = xw * dinv_ref[...]


def _dense1_call(x, W1, dinv):
    blk = 1000
    return pl.pallas_call(
        _k_dense1_body,
        grid=(N // blk,),
        in_specs=[
            pl.BlockSpec((blk, D_IN), lambda i: (i, 0)),
            pl.BlockSpec((D_IN, H), lambda i: (0, 0)),
            pl.BlockSpec((blk, H), lambda i: (i, 0)),
        ],
        out_specs=pl.BlockSpec((blk, H), lambda i: (i, 0)),
        out_shape=jax.ShapeDtypeStruct((N, H), jnp.float32),
    )(x, W1, dinv)


def _k_dense2_body(acc_ref, xs1_ref, dinv_ref, b1_ref, w2_ref, xs2_ref):
    a = acc_ref[...]
    dinv = dinv_ref[...]
    h = dinv * (a[0] + a[1] + xs1_ref[...]) + b1_ref[...]
    h = jnp.maximum(h, 0.0)
    xs2_ref[...] = jnp.dot(h, w2_ref[...],
                           preferred_element_type=jnp.float32) * dinv


def _dense2_call(acc, xs1, dinv, b1, W2):
    blk = 1000
    return pl.pallas_call(
        _k_dense2_body,
        grid=(N // blk,),
        in_specs=[
            pl.BlockSpec((NC, blk, H), lambda i: (0, i, 0)),
            pl.BlockSpec((blk, H), lambda i: (i, 0)),
            pl.BlockSpec((blk, H), lambda i: (i, 0)),
            pl.BlockSpec((1, H), lambda i: (0, 0)),
            pl.BlockSpec((H, H), lambda i: (0, 0)),
        ],
        out_specs=pl.BlockSpec((blk, H), lambda i: (i, 0)),
        out_shape=jax.ShapeDtypeStruct((N, H), jnp.float32),
    )(acc, xs1, dinv, b1, W2)


def _k_dense3_body(acc_ref, xs2_ref, dinv_ref, b2_ref, out_ref):
    a = acc_ref[...]
    out_ref[...] = dinv_ref[...] * (a[0] + a[1] + xs2_ref[...]) + b2_ref[...]


def _dense3_call(acc, xs2, dinv, b2):
    blk = 1000
    return pl.pallas_call(
        _k_dense3_body,
        grid=(N // blk,),
        in_specs=[
            pl.BlockSpec((NC, blk, H), lambda i: (0, i, 0)),
            pl.BlockSpec((blk, H), lambda i: (i, 0)),
            pl.BlockSpec((blk, H), lambda i: (i, 0)),
            pl.BlockSpec((1, H), lambda i: (0, 0)),
        ],
        out_specs=pl.BlockSpec((blk, H), lambda i: (i, 0)),
        out_shape=jax.ShapeDtypeStruct((N, H), jnp.float32),
    )(acc, xs2, dinv, b2)


def kernel(x, coo, W1, b1, W2, b2):
    src = coo[:, 0]
    dst = coo[:, 1]
    npad = EPAD - E
    src_p = jnp.concatenate([src, jnp.zeros((npad,), jnp.int32)])
    dst_p = jnp.concatenate([dst, jnp.full((npad,), N, jnp.int32)])
    sd2d = jnp.stack([src_p.reshape(NROWS, 128),
                      dst_p.reshape(NROWS, 128)], axis=1)

    deg_parts = _k_deg(sd2d)
    dinv = _dinv_call(deg_parts)

    xs1 = _dense1_call(x, W1, dinv)
    acc1 = _k_scat(xs1, sd2d)
    xs2 = _dense2_call(acc1, xs1, dinv, b1.reshape(1, H), W2)
    acc2 = _k_scat(xs2, sd2d)
    out = _dense3_call(acc2, xs2, dinv, b2.reshape(1, H))
    return out
